# Initial kernel scaffold; baseline (speedup 1.0000x reference)
import jax
import jax.numpy as jnp
from jax.experimental import pallas as pl


def kernel(node_type, edge_type, edge_index, scorer_weight):
    raise NotImplementedError("write your pallas kernel here")


# speedup vs baseline: 668.8162x; 668.8162x over previous
import functools

import jax
import jax.numpy as jnp
from jax import lax
from jax.experimental import pallas as pl
from jax.experimental.pallas import tpu as pltpu
from jax.experimental.pallas import tpu_sc as plsc

N_NODES = 100_000
N_EDGES = 6_400_000
TABLE_ROWS = 576
NUM_WORKERS = 32
EDGES_PER_WORKER = N_EDGES // NUM_WORKERS
CHUNK = 4_000
NUM_CHUNKS = EDGES_PER_WORKER // CHUNK
L = 16


def _sc_graph_counter(node_type, edge_type, edge_index, w_flat):
    mesh = plsc.VectorSubcoreMesh(core_axis_name="c", subcore_axis_name="s")

    @functools.partial(
        pl.kernel,
        mesh=mesh,
        out_type=jax.ShapeDtypeStruct((NUM_WORKERS, L), jnp.float32),
        compiler_params=pltpu.CompilerParams(needs_layout_passes=False),
        scratch_types=[
            pltpu.VMEM((N_NODES,), jnp.int32),
            pltpu.VMEM((TABLE_ROWS,), jnp.float32),
            pltpu.VMEM((CHUNK,), jnp.int32),
            pltpu.VMEM((CHUNK,), jnp.int32),
            pltpu.VMEM((CHUNK,), jnp.int32),
            pltpu.VMEM((L,), jnp.float32),
        ],
    )
    def k(nt_hbm, et_hbm, ei_hbm, w_hbm, out_hbm,
          nt_v, w_v, et_v, src_v, dst_v, acc_v):
        wid = lax.axis_index("s") * 2 + lax.axis_index("c")
        pltpu.sync_copy(nt_hbm, nt_v)
        pltpu.sync_copy(w_hbm, w_v)
        base_w = wid * EDGES_PER_WORKER

        def chunk_body(ci, acc):
            base = base_w + ci * CHUNK
            pltpu.sync_copy(et_hbm.at[pl.ds(base, CHUNK)], et_v)
            pltpu.sync_copy(ei_hbm.at[pl.ds(base, CHUNK)], src_v)
            pltpu.sync_copy(ei_hbm.at[pl.ds(N_EDGES + base, CHUNK)], dst_v)

            def vbody(i, acc):
                sl = pl.ds(i * L, L)
                et = et_v[sl]
                s = src_v[sl]
                d = dst_v[sl]
                a = plsc.load_gather(nt_v, [s])
                b = plsc.load_gather(nt_v, [d])
                enc = et * 9 + a * 3 + b
                wv = plsc.load_gather(w_v, [enc])
                return acc + wv

            return lax.fori_loop(0, CHUNK // L, vbody, acc)

        acc = lax.fori_loop(0, NUM_CHUNKS, chunk_body,
                            jnp.zeros((L,), jnp.float32))
        acc_v[...] = acc
        pltpu.sync_copy(acc_v, out_hbm.at[wid])

    return k(node_type, edge_type, edge_index, w_flat)


def kernel(node_type, edge_type, edge_index, scorer_weight):
    w_flat = scorer_weight.reshape(TABLE_ROWS)
    ei_flat = edge_index.reshape(2 * N_EDGES)
    partials = _sc_graph_counter(node_type, edge_type, ei_flat, w_flat)
    return jnp.sum(partials).reshape(1, 1)


# speedup vs baseline: 1337.5030x; 1.9998x over previous
import functools

import jax
import jax.numpy as jnp
from jax import lax
from jax.experimental import pallas as pl
from jax.experimental.pallas import tpu as pltpu
from jax.experimental.pallas import tpu_sc as plsc

N_NODES = 100_000
N_EDGES = 6_400_000
TABLE_ROWS = 576
NUM_WORKERS = 32
EDGES_PER_WORKER = N_EDGES // NUM_WORKERS
CHUNK = 4_000
NUM_CHUNKS = EDGES_PER_WORKER // CHUNK
L = 16
UNROLL = 8


def _sc_graph_counter(node_type, edge_type, edge_index, w_flat):
    mesh = plsc.VectorSubcoreMesh(core_axis_name="c", subcore_axis_name="s")

    @functools.partial(
        pl.kernel,
        mesh=mesh,
        out_type=jax.ShapeDtypeStruct((NUM_WORKERS, L), jnp.float32),
        compiler_params=pltpu.CompilerParams(needs_layout_passes=False),
        scratch_types=[
            pltpu.VMEM((N_NODES,), jnp.int32),
            pltpu.VMEM((TABLE_ROWS,), jnp.float32),
            pltpu.VMEM((CHUNK,), jnp.int32),
            pltpu.VMEM((CHUNK,), jnp.int32),
            pltpu.VMEM((CHUNK,), jnp.int32),
            pltpu.VMEM((CHUNK,), jnp.int32),
            pltpu.VMEM((CHUNK,), jnp.int32),
            pltpu.VMEM((CHUNK,), jnp.int32),
            pltpu.VMEM((L,), jnp.float32),
            pltpu.SemaphoreType.DMA,
            pltpu.SemaphoreType.DMA,
        ],
    )
    def k(nt_hbm, et_hbm, ei_hbm, w_hbm, out_hbm,
          nt_v, w_v, et0, et1, src0, src1, dst0, dst1, acc_v, sem0, sem1):
        wid = lax.axis_index("s") * 2 + lax.axis_index("c")
        pltpu.sync_copy(nt_hbm, nt_v)
        pltpu.sync_copy(w_hbm, w_v)
        base_w = wid * EDGES_PER_WORKER
        sems = (sem0, sem1)
        bufs = ((et0, src0, dst0), (et1, src1, dst1))

        def start(ci, b):
            ci = jnp.minimum(ci, NUM_CHUNKS - 1)
            base = base_w + ci * CHUNK
            et_b, src_b, dst_b = bufs[b]
            pltpu.async_copy(et_hbm.at[pl.ds(base, CHUNK)], et_b, sems[b])
            pltpu.async_copy(ei_hbm.at[pl.ds(base, CHUNK)], src_b, sems[b])
            pltpu.async_copy(ei_hbm.at[pl.ds(N_EDGES + base, CHUNK)],
                             dst_b, sems[b])

        def wait(b):
            et_b, src_b, dst_b = bufs[b]
            pltpu.make_async_copy(et_hbm.at[pl.ds(0, CHUNK)], et_b,
                                  sems[b]).wait()
            pltpu.make_async_copy(ei_hbm.at[pl.ds(0, CHUNK)], src_b,
                                  sems[b]).wait()
            pltpu.make_async_copy(ei_hbm.at[pl.ds(0, CHUNK)], dst_b,
                                  sems[b]).wait()

        def compute(b, acc):
            et_b, src_b, dst_b = bufs[b]

            def vbody(i, acc):
                sl = pl.ds(i * L, L)
                et = et_b[sl]
                s = src_b[sl]
                d = dst_b[sl]
                a = plsc.load_gather(nt_v, [s])
                bb = plsc.load_gather(nt_v, [d])
                enc = et * 9 + a * 3 + bb
                wv = plsc.load_gather(w_v, [enc])
                return acc + wv

            return plsc.parallel_loop(
                0, CHUNK // L, carry=acc, unroll=UNROLL)(vbody)

        start(0, 0)
        start(1, 1)

        def pair_body(kk, acc):
            ci = kk * 2
            wait(0)
            acc = compute(0, acc)
            start(ci + 2, 0)
            wait(1)
            acc = compute(1, acc)
            start(ci + 3, 1)
            return acc

        acc = lax.fori_loop(0, NUM_CHUNKS // 2, pair_body,
                            jnp.zeros((L,), jnp.float32))
        wait(0)
        wait(1)
        acc_v[...] = acc
        pltpu.sync_copy(acc_v, out_hbm.at[wid])

    return k(node_type, edge_type, edge_index, w_flat)


def kernel(node_type, edge_type, edge_index, scorer_weight):
    w_flat = scorer_weight.reshape(TABLE_ROWS)
    ei_flat = edge_index.reshape(2 * N_EDGES)
    partials = _sc_graph_counter(node_type, edge_type, ei_flat, w_flat)
    return jnp.sum(partials).reshape(1, 1)


# speedup vs baseline: 1363.8933x; 1.0197x over previous
import functools

import jax
import jax.numpy as jnp
from jax import lax
from jax.experimental import pallas as pl
from jax.experimental.pallas import tpu as pltpu
from jax.experimental.pallas import tpu_sc as plsc

N_NODES = 100_000
N_EDGES = 6_400_000
TABLE_ROWS = 576
NUM_WORKERS = 32
EDGES_PER_WORKER = N_EDGES // NUM_WORKERS
CHUNK = 4_000
NUM_CHUNKS = EDGES_PER_WORKER // CHUNK
L = 16
NACC = 5
UNROLL = 2


def _sc_graph_counter(node_type, edge_type, edge_index, w_flat):
    mesh = plsc.VectorSubcoreMesh(core_axis_name="c", subcore_axis_name="s")

    @functools.partial(
        pl.kernel,
        mesh=mesh,
        out_type=jax.ShapeDtypeStruct((NUM_WORKERS, L), jnp.float32),
        compiler_params=pltpu.CompilerParams(needs_layout_passes=False),
        scratch_types=[
            pltpu.VMEM((N_NODES,), jnp.int32),
            pltpu.VMEM((TABLE_ROWS,), jnp.float32),
            pltpu.VMEM((CHUNK,), jnp.int32),
            pltpu.VMEM((CHUNK,), jnp.int32),
            pltpu.VMEM((CHUNK,), jnp.int32),
            pltpu.VMEM((CHUNK,), jnp.int32),
            pltpu.VMEM((CHUNK,), jnp.int32),
            pltpu.VMEM((CHUNK,), jnp.int32),
            pltpu.VMEM((L,), jnp.float32),
            pltpu.SemaphoreType.DMA,
            pltpu.SemaphoreType.DMA,
        ],
    )
    def k(nt_hbm, et_hbm, ei_hbm, w_hbm, out_hbm,
          nt_v, w_v, et0, et1, src0, src1, dst0, dst1, acc_v,
          sem0, sem1):
        sid = lax.axis_index("s")
        wid = sid * 2 + lax.axis_index("c")
        base_w = wid * EDGES_PER_WORKER
        sems = (sem0, sem1)
        bufs = ((et0, src0, dst0), (et1, src1, dst1))

        def start(ci, b):
            ci = jnp.minimum(ci, NUM_CHUNKS - 1)
            base = base_w + ci * CHUNK
            et_b, src_b, dst_b = bufs[b]
            pltpu.async_copy(et_hbm.at[pl.ds(base, CHUNK)], et_b, sems[b])
            pltpu.async_copy(ei_hbm.at[pl.ds(base, CHUNK)], src_b, sems[b])
            pltpu.async_copy(ei_hbm.at[pl.ds(N_EDGES + base, CHUNK)],
                             dst_b, sems[b])

        def wait(b):
            et_b, src_b, dst_b = bufs[b]
            pltpu.make_async_copy(et_hbm.at[pl.ds(0, CHUNK)], et_b,
                                  sems[b]).wait()
            pltpu.make_async_copy(ei_hbm.at[pl.ds(0, CHUNK)], src_b,
                                  sems[b]).wait()
            pltpu.make_async_copy(ei_hbm.at[pl.ds(0, CHUNK)], dst_b,
                                  sems[b]).wait()

        start(0, 0)
        start(1, 1)

        pltpu.sync_copy(nt_hbm, nt_v)
        pltpu.sync_copy(w_hbm, w_v)

        def compute(b, accs):
            et_b, src_b, dst_b = bufs[b]

            def vbody(i, accs):
                out = []
                for u in range(NACC):
                    sl = pl.ds((i * NACC + u) * L, L)
                    et = et_b[sl]
                    s = src_b[sl]
                    d = dst_b[sl]
                    a = plsc.load_gather(nt_v, [s])
                    bb = plsc.load_gather(nt_v, [d])
                    enc = et * 9 + a * 3 + bb
                    wv = plsc.load_gather(w_v, [enc])
                    out.append(accs[u] + wv)
                return tuple(out)

            return plsc.parallel_loop(
                0, (CHUNK // L) // NACC, carry=accs, unroll=UNROLL)(vbody)

        def pair_body(kk, accs):
            ci = kk * 2
            wait(0)
            accs = compute(0, accs)
            start(ci + 2, 0)
            wait(1)
            accs = compute(1, accs)
            start(ci + 3, 1)
            return accs

        zero = jnp.zeros((L,), jnp.float32)
        accs = lax.fori_loop(0, NUM_CHUNKS // 2, pair_body, (zero,) * NACC)
        wait(0)
        wait(1)
        acc = accs[0]
        for u in range(1, NACC):
            acc = acc + accs[u]
        acc_v[...] = acc
        pltpu.sync_copy(acc_v, out_hbm.at[wid])

    return k(node_type, edge_type, edge_index, w_flat)


def kernel(node_type, edge_type, edge_index, scorer_weight):
    w_flat = scorer_weight.reshape(TABLE_ROWS)
    ei_flat = edge_index.reshape(2 * N_EDGES)
    partials = _sc_graph_counter(node_type, edge_type, ei_flat, w_flat)
    return jnp.sum(partials).reshape(1, 1)


# speedup vs baseline: 1366.7909x; 1.0021x over previous
import functools

import jax
import jax.numpy as jnp
from jax import lax
from jax.experimental import pallas as pl
from jax.experimental.pallas import tpu as pltpu
from jax.experimental.pallas import tpu_sc as plsc

N_NODES = 100_000
N_EDGES = 6_400_000
TABLE_ROWS = 576
NUM_WORKERS = 32
EDGES_PER_WORKER = N_EDGES // NUM_WORKERS
CHUNK = 4_000
NUM_CHUNKS = EDGES_PER_WORKER // CHUNK
L = 16
NACC = 5
UNROLL = 2


def _sc_graph_counter(node_type, edge_type, edge_index, w_flat):
    mesh = plsc.VectorSubcoreMesh(core_axis_name="c", subcore_axis_name="s")

    @functools.partial(
        pl.kernel,
        mesh=mesh,
        out_type=jax.ShapeDtypeStruct((NUM_WORKERS, L), jnp.float32),
        compiler_params=pltpu.CompilerParams(
            needs_layout_passes=False, use_tc_tiling_on_sc=False),
        scratch_types=[
            pltpu.VMEM((N_NODES,), jnp.int32),
            pltpu.VMEM((TABLE_ROWS,), jnp.float32),
            pltpu.VMEM((CHUNK,), jnp.int32),
            pltpu.VMEM((CHUNK,), jnp.int32),
            pltpu.VMEM((CHUNK,), jnp.int32),
            pltpu.VMEM((CHUNK,), jnp.int32),
            pltpu.VMEM((CHUNK,), jnp.int32),
            pltpu.VMEM((CHUNK,), jnp.int32),
            pltpu.VMEM((L,), jnp.float32),
            pltpu.SemaphoreType.DMA,
            pltpu.SemaphoreType.DMA,
        ],
    )
    def k(nt_hbm, et_hbm, ei_hbm, w_hbm, out_hbm,
          nt_v, w_v, et0, et1, src0, src1, dst0, dst1, acc_v,
          sem0, sem1):
        sid = lax.axis_index("s")
        wid = sid * 2 + lax.axis_index("c")
        base_w = wid * EDGES_PER_WORKER
        sems = (sem0, sem1)
        bufs = ((et0, src0, dst0), (et1, src1, dst1))

        def start(ci, b):
            ci = jnp.minimum(ci, NUM_CHUNKS - 1)
            base = base_w + ci * CHUNK
            et_b, src_b, dst_b = bufs[b]
            pltpu.async_copy(et_hbm.at[pl.ds(base, CHUNK)], et_b, sems[b])
            pltpu.async_copy(ei_hbm.at[0, pl.ds(base, CHUNK)], src_b, sems[b])
            pltpu.async_copy(ei_hbm.at[1, pl.ds(base, CHUNK)], dst_b, sems[b])

        def wait(b):
            et_b, src_b, dst_b = bufs[b]
            pltpu.make_async_copy(et_hbm.at[pl.ds(0, CHUNK)], et_b,
                                  sems[b]).wait()
            pltpu.make_async_copy(ei_hbm.at[0, pl.ds(0, CHUNK)], src_b,
                                  sems[b]).wait()
            pltpu.make_async_copy(ei_hbm.at[1, pl.ds(0, CHUNK)], dst_b,
                                  sems[b]).wait()

        start(0, 0)
        start(1, 1)

        pltpu.sync_copy(nt_hbm, nt_v)
        pltpu.sync_copy(w_hbm, w_v)

        def compute(b, accs):
            et_b, src_b, dst_b = bufs[b]

            def vbody(i, accs):
                out = []
                for u in range(NACC):
                    sl = pl.ds((i * NACC + u) * L, L)
                    et = et_b[sl]
                    s = src_b[sl]
                    d = dst_b[sl]
                    a = plsc.load_gather(nt_v, [s])
                    bb = plsc.load_gather(nt_v, [d])
                    enc = et * 9 + a * 3 + bb
                    wv = plsc.load_gather(w_v, [enc])
                    out.append(accs[u] + wv)
                return tuple(out)

            return plsc.parallel_loop(
                0, (CHUNK // L) // NACC, carry=accs, unroll=UNROLL)(vbody)

        def pair_body(kk, accs):
            ci = kk * 2
            wait(0)
            accs = compute(0, accs)
            start(ci + 2, 0)
            wait(1)
            accs = compute(1, accs)
            start(ci + 3, 1)
            return accs

        zero = jnp.zeros((L,), jnp.float32)
        accs = lax.fori_loop(0, NUM_CHUNKS // 2, pair_body, (zero,) * NACC)
        wait(0)
        wait(1)
        acc = accs[0]
        for u in range(1, NACC):
            acc = acc + accs[u]
        acc_v[...] = acc
        pltpu.sync_copy(acc_v, out_hbm.at[wid])

    return k(node_type, edge_type, edge_index, w_flat)


def kernel(node_type, edge_type, edge_index, scorer_weight):
    w_flat = scorer_weight.reshape(TABLE_ROWS)
    partials = _sc_graph_counter(node_type, edge_type, edge_index, w_flat)
    return jnp.sum(partials).reshape(1, 1)


# speedup vs baseline: 1672.0190x; 1.2233x over previous
import functools

import jax
import jax.numpy as jnp
from jax import lax
from jax.experimental import pallas as pl
from jax.experimental.pallas import tpu as pltpu
from jax.experimental.pallas import tpu_sc as plsc

N_NODES = 100_000
N_EDGES = 6_400_000
TABLE_ROWS = 576
NUM_WORKERS = 32
CHUNK = 2_048
NUM_CHUNKS = N_EDGES // CHUNK
SLOTS = 98
L = 16
NACC = 4
UNROLL = 2


def _sc_graph_counter(node_type, edge_type, edge_index, w_flat):
    mesh = plsc.VectorSubcoreMesh(core_axis_name="c", subcore_axis_name="s")

    @functools.partial(
        pl.kernel,
        mesh=mesh,
        out_type=jax.ShapeDtypeStruct((NUM_WORKERS, L), jnp.float32),
        compiler_params=pltpu.CompilerParams(needs_layout_passes=False),
        scratch_types=[
            pltpu.VMEM((N_NODES,), jnp.int32),
            pltpu.VMEM((TABLE_ROWS,), jnp.float32),
            pltpu.VMEM((CHUNK,), jnp.int32),
            pltpu.VMEM((CHUNK,), jnp.int32),
            pltpu.VMEM((2, CHUNK), jnp.int32),
            pltpu.VMEM((2, CHUNK), jnp.int32),
            pltpu.VMEM((L,), jnp.float32),
            pltpu.SemaphoreType.DMA,
            pltpu.SemaphoreType.DMA,
        ],
    )
    def k(nt_hbm, et_hbm, ei_hbm, w_hbm, out_hbm,
          nt_v, w_v, et0, et1, sd0, sd1, acc_v, sem0, sem1):
        wid = lax.axis_index("s") * 2 + lax.axis_index("c")
        pltpu.sync_copy(nt_hbm, nt_v)
        pltpu.sync_copy(w_hbm, w_v)
        sems = (sem0, sem1)
        bufs = ((et0, sd0), (et1, sd1))

        def chunk_id(slot):
            return jnp.minimum(wid + slot * NUM_WORKERS, NUM_CHUNKS - 1)

        def start(slot, b):
            base = chunk_id(slot) * CHUNK
            et_b, sd_b = bufs[b]
            pltpu.async_copy(et_hbm.at[pl.ds(base, CHUNK)], et_b, sems[b])
            pltpu.async_copy(ei_hbm.at[:, pl.ds(base, CHUNK)], sd_b, sems[b])

        def wait(b):
            et_b, sd_b = bufs[b]
            pltpu.make_async_copy(et_hbm.at[pl.ds(0, CHUNK)], et_b,
                                  sems[b]).wait()
            pltpu.make_async_copy(ei_hbm.at[:, pl.ds(0, CHUNK)], sd_b,
                                  sems[b]).wait()

        lane = lax.iota(jnp.int32, L)

        def compute(slot, b, accs):
            et_b, sd_b = bufs[b]
            valid = (chunk_id(slot) == wid + slot * NUM_WORKERS)
            vmask = jnp.where(valid, 1.0, 0.0).astype(jnp.float32)
            vmask = jnp.broadcast_to(vmask, (L,))

            def vbody(i, accs):
                out = []
                for u in range(NACC):
                    j = i * NACC + u
                    sl = pl.ds(j * L, L)
                    col = j * L + lane
                    et = et_b[sl]
                    s = plsc.load_gather(sd_b, [jnp.zeros((L,), jnp.int32),
                                                col])
                    d = plsc.load_gather(sd_b, [jnp.ones((L,), jnp.int32),
                                                col])
                    a = plsc.load_gather(nt_v, [s])
                    bb = plsc.load_gather(nt_v, [d])
                    enc = et * 9 + a * 3 + bb
                    wv = plsc.load_gather(w_v, [enc])
                    out.append(accs[u] + wv * vmask)
                return tuple(out)

            return plsc.parallel_loop(
                0, (CHUNK // L) // NACC, carry=accs, unroll=UNROLL)(vbody)

        start(0, 0)
        start(1, 1)

        def pair_body(kk, accs):
            slot = kk * 2
            wait(0)
            accs = compute(slot, 0, accs)
            start(slot + 2, 0)
            wait(1)
            accs = compute(slot + 1, 1, accs)
            start(slot + 3, 1)
            return accs

        zero = jnp.zeros((L,), jnp.float32)
        accs = lax.fori_loop(0, SLOTS // 2, pair_body, (zero,) * NACC)
        wait(0)
        wait(1)
        acc = accs[0]
        for u in range(1, NACC):
            acc = acc + accs[u]
        acc_v[...] = acc
        pltpu.sync_copy(acc_v, out_hbm.at[wid])

    return k(node_type, edge_type, edge_index, w_flat)


def kernel(node_type, edge_type, edge_index, scorer_weight):
    w_flat = scorer_weight.reshape(TABLE_ROWS)
    partials = _sc_graph_counter(node_type, edge_type, edge_index, w_flat)
    return jnp.sum(partials).reshape(1, 1)
